# Initial kernel scaffold; baseline (speedup 1.0000x reference)
import jax
import jax.numpy as jnp
from jax.experimental import pallas as pl


def kernel(image, mat_z, psf_vals, psf_rows, psf_cols):
    raise NotImplementedError("write your pallas kernel here")


# speedup vs baseline: 6.1223x; 6.1223x over previous
import functools

import jax
import jax.numpy as jnp
from jax import lax
from jax.experimental import pallas as pl
from jax.experimental.pallas import tpu as pltpu
from jax.experimental.pallas import tpu_sc as plsc

NX, NY, NZ = 256, 256, 128
N = NX * NY
NNZ = 4194304

NC, NS, L = 2, 16, 16
CH = 8192
NCHUNK = N // CH
CPC = NCHUNK // NC
K = 112
CAP = 128
BB = 4096
E = NNZ // NS
NBLK = E // BB
STRIPE = CH // NS
ZR = 128


def _sc_scatter(squ, vals, rows, cols):
    mesh = plsc.VectorSubcoreMesh(
        core_axis_name="c", subcore_axis_name="s",
        num_cores=NC, num_subcores=NS)

    @functools.partial(
        pl.kernel,
        out_type=jax.ShapeDtypeStruct((N, NZ), jnp.float32),
        mesh=mesh,
        scratch_types=[
            pltpu.VMEM_SHARED((CH, NZ), jnp.float32),
            pltpu.VMEM((BB,), jnp.int32),
            pltpu.VMEM((BB,), jnp.int32),
            pltpu.VMEM((BB,), jnp.float32),
            pltpu.VMEM((CAP,), jnp.int32),
            pltpu.VMEM((CAP,), jnp.int32),
            pltpu.VMEM((CAP,), jnp.float32),
            pltpu.VMEM((K,), jnp.int32),
            pltpu.VMEM((K,), jnp.int32),
            pltpu.VMEM((K,), jnp.float32),
            pltpu.VMEM((K, NZ), jnp.float32),
            pltpu.VMEM((ZR, NZ), jnp.float32),
        ],
        compiler_params=pltpu.CompilerParams(needs_layout_passes=False),
    )
    def scatter_kernel(squ_hbm, vals_hbm, rows_hbm, cols_hbm, out_hbm,
                       acc, rblk, cblk, vblk, pend_r, pend_c, pend_v,
                       fr, fc, fv, rowbuf, zbuf):
        cid = lax.axis_index("c")
        sid = lax.axis_index("s")
        ebase = sid * E
        lanes = lax.iota(jnp.int32, L)
        zvec = jnp.zeros((L,), jnp.float32)

        def zb(t, carry):
            zbuf[t // (NZ // L), pl.ds((t % (NZ // L)) * L, L)] = zvec
            return carry
        lax.fori_loop(0, ZR * (NZ // L), zb, 0)

        def flush():
            def cp(t, carry):
                sl = pl.ds(t * L, L)
                fr[sl] = pend_r[sl]
                fc[sl] = pend_c[sl]
                fv[sl] = pend_v[sl]
                return carry
            lax.fori_loop(0, K // L, cp, 0)
            pltpu.sync_copy(squ_hbm.at[fr], rowbuf)
            def scale16(t, carry):
                vv = fv[pl.ds(t * L, L)]
                for lane in range(L):
                    k = t * L + lane
                    v = vv[lane]
                    for g in range(NZ // L):
                        sl = pl.ds(g * L, L)
                        rowbuf[k, sl] = rowbuf[k, sl] * v
                return carry
            lax.fori_loop(0, K // L, scale16, 0)
            pltpu.sync_copy(rowbuf, acc.at[fc], add=True)

        def chunk_body(j, carry):
            base = (cid * CPC + j) * CH
            def z(i, c2):
                pltpu.sync_copy(zbuf, acc.at[pl.ds(sid * STRIPE + i * ZR, ZR)])
                return c2
            lax.fori_loop(0, STRIPE // ZR, z, 0)
            plsc.subcore_barrier()

            def blk_body(b, cnt):
                off = ebase + b * BB
                pltpu.sync_copy(rows_hbm.at[pl.ds(off, BB)], rblk)
                pltpu.sync_copy(cols_hbm.at[pl.ds(off, BB)], cblk)
                pltpu.sync_copy(vals_hbm.at[pl.ds(off, BB)], vblk)
                def grp(g, cnt):
                    sl = pl.ds(g * L, L)
                    cvec = cblk[sl]
                    m = (cvec >= base) & (cvec < base + CH)
                    dst = pl.ds(cnt, L)
                    plsc.store_compressed(pend_c.at[dst], cvec - base, mask=m)
                    plsc.store_compressed(pend_r.at[dst], rblk[sl], mask=m)
                    plsc.store_compressed(pend_v.at[dst], vblk[sl], mask=m)
                    cnt = cnt + jnp.sum(m.astype(jnp.int32))
                    @pl.when(cnt >= K)
                    def _():
                        flush()
                        tsl = pl.ds(K, L)
                        hsl = pl.ds(0, L)
                        pend_r[hsl] = pend_r[tsl]
                        pend_c[hsl] = pend_c[tsl]
                        pend_v[hsl] = pend_v[tsl]
                    return jnp.where(cnt >= K, cnt - K, cnt)
                return lax.fori_loop(0, BB // L, grp, cnt)

            cnt = lax.fori_loop(0, NBLK, blk_body, jnp.int32(0))

            def san(t, c2):
                gl = lanes + t * L
                m = gl < cnt
                sl = pl.ds(t * L, L)
                pend_c[sl] = jnp.where(m, pend_c[sl], 0)
                pend_r[sl] = jnp.where(m, pend_r[sl], 0)
                pend_v[sl] = jnp.where(m, pend_v[sl], jnp.float32(0.0))
                return c2
            lax.fori_loop(0, K // L, san, 0)
            flush()

            plsc.subcore_barrier()
            def co(i, c2):
                r0 = sid * STRIPE + i * ZR
                pltpu.sync_copy(acc.at[pl.ds(r0, ZR)],
                                out_hbm.at[pl.ds(base + r0, ZR)])
                return c2
            lax.fori_loop(0, STRIPE // ZR, co, 0)
            return carry

        lax.fori_loop(0, CPC, chunk_body, 0)

    return scatter_kernel(squ, vals, rows, cols)


def _tc_matmul(tmp, mat_z):
    BM = 2048

    def mm(x_ref, w_ref, o_ref):
        o_ref[...] = lax.dot_general(
            x_ref[...], w_ref[...], (((1,), (1,)), ((), ())),
            preferred_element_type=jnp.float32)

    return pl.pallas_call(
        mm,
        grid=(N // BM,),
        in_specs=[pl.BlockSpec((BM, NZ), lambda i: (i, 0)),
                  pl.BlockSpec((NZ, NZ), lambda i: (0, 0))],
        out_specs=pl.BlockSpec((BM, NZ), lambda i: (i, 0)),
        out_shape=jax.ShapeDtypeStruct((N, NZ), jnp.float32),
    )(tmp, mat_z)


def kernel(image, mat_z, psf_vals, psf_rows, psf_cols):
    squ = image.reshape(N, NZ)
    tmp = _sc_scatter(squ, psf_vals, psf_rows, psf_cols)
    out = _tc_matmul(tmp, mat_z)
    return out.reshape(NX, NY, NZ)
